# Initial kernel scaffold; baseline (speedup 1.0000x reference)
import jax
import jax.numpy as jnp
from jax.experimental import pallas as pl


def kernel(x, edge_index, W1, b1, W2, b2, W3, b3):
    raise NotImplementedError("write your pallas kernel here")


# speedup vs baseline: 29.3264x; 29.3264x over previous
import functools

import jax
import jax.numpy as jnp
from jax import lax
from jax.experimental import pallas as pl
from jax.experimental.pallas import tpu as pltpu
from jax.experimental.pallas import tpu_sc as plsc

N = 10000
E = 320000
FIN = 128
D = 16
NC, NS = 2, 16
NW = NC * NS
CH = 128
K = 80
EPAD = NW * K * CH
R = 10240
ZRD = R // NS
BR = 512


def _mesh():
    return plsc.VectorSubcoreMesh(core_axis_name="c", subcore_axis_name="s")


_SC_PARAMS = pltpu.CompilerParams(use_tc_tiling_on_sc=False)


def _sc_degree(dst3):
    @functools.partial(
        pl.kernel,
        mesh=_mesh(),
        out_type=jax.ShapeDtypeStruct((NC, R), jnp.float32),
        compiler_params=_SC_PARAMS,
        scratch_types=[
            pltpu.VMEM((K, CH), jnp.int32),
            pltpu.VMEM((CH,), jnp.float32),
            pltpu.VMEM((ZRD,), jnp.float32),
            pltpu.VMEM_SHARED((R,), jnp.float32),
        ],
    )
    def k(dst_hbm, out_hbm, dst_v, ones_v, zb_v, dtab):
        c = lax.axis_index("c")
        s = lax.axis_index("s")
        w = c * NS + s

        z16 = jnp.zeros((16,), jnp.float32)
        o16 = jnp.ones((16,), jnp.float32)

        def fz(i, carry):
            zb_v[pl.ds(i * 16, 16)] = z16
            return carry

        lax.fori_loop(0, ZRD // 16, fz, 0)

        def fo(i, carry):
            ones_v[pl.ds(i * 16, 16)] = o16
            return carry

        lax.fori_loop(0, CH // 16, fo, 0)

        pltpu.sync_copy(zb_v, dtab.at[pl.ds(s * ZRD, ZRD)])
        plsc.subcore_barrier()

        pltpu.sync_copy(dst_hbm.at[w], dst_v)

        def step(j, carry):
            pltpu.sync_copy(ones_v, dtab.at[dst_v.at[j]], add=True)
            return carry

        lax.fori_loop(0, K, step, 0)
        plsc.subcore_barrier()

        @pl.when(s == 0)
        def _():
            pltpu.sync_copy(dtab, out_hbm.at[c])

    return k(dst3)


def _sc_aggregate(ytab, src3, dst3):
    @functools.partial(
        pl.kernel,
        mesh=_mesh(),
        out_type=jax.ShapeDtypeStruct((NC, R, D), jnp.float32),
        compiler_params=_SC_PARAMS,
        scratch_types=[
            pltpu.VMEM((K, CH), jnp.int32),
            pltpu.VMEM((K, CH), jnp.int32),
            pltpu.VMEM((CH, D), jnp.float32),
            pltpu.VMEM((ZRD, D), jnp.float32),
            pltpu.VMEM_SHARED((R, D), jnp.float32),
            pltpu.SemaphoreType.DMA,
        ],
    )
    def k(ytab_hbm, src_hbm, dst_hbm, out_hbm, src_v, dst_v, rows, zb, ztab, sem):
        c = lax.axis_index("c")
        s = lax.axis_index("s")
        w = c * NS + s

        zrow = jnp.zeros((D,), jnp.float32)

        def fz(i, carry):
            zb[i, :] = zrow
            return carry

        lax.fori_loop(0, ZRD, fz, 0)
        pltpu.sync_copy(zb, ztab.at[pl.ds(s * ZRD, ZRD)])
        plsc.subcore_barrier()

        pltpu.sync_copy(src_hbm.at[w], src_v)
        pltpu.sync_copy(dst_hbm.at[w], dst_v)

        def step(j, carry):
            pltpu.async_copy(ytab_hbm.at[src_v.at[j]], rows, sem).wait()
            pltpu.sync_copy(rows, ztab.at[dst_v.at[j]], add=True)
            return carry

        lax.fori_loop(0, K, step, 0)
        plsc.subcore_barrier()

        @pl.when(s == 0)
        def _():
            pltpu.sync_copy(ztab, out_hbm.at[c])

    return k(ytab, src3, dst3)


def _tc_prep(deg2, xp, w1):
    def body(deg_ref, x_ref, w1_ref, dinv_ref, u1_ref):
        deg = deg_ref[0, :] + deg_ref[1, :] + 1.0
        dinv = lax.rsqrt(deg)
        lin = jnp.dot(x_ref[...], w1_ref[...], preferred_element_type=jnp.float32)
        dcol = dinv[:, None]
        u1_ref[...] = lin * dcol
        dinv_ref[...] = jnp.broadcast_to(dcol, (BR, D))

    return pl.pallas_call(
        body,
        grid=(R // BR,),
        in_specs=[
            pl.BlockSpec((NC, BR), lambda i: (0, i)),
            pl.BlockSpec((BR, FIN), lambda i: (i, 0)),
            pl.BlockSpec((FIN, D), lambda i: (0, 0)),
        ],
        out_specs=[
            pl.BlockSpec((BR, D), lambda i: (i, 0)),
            pl.BlockSpec((BR, D), lambda i: (i, 0)),
        ],
        out_shape=[
            jax.ShapeDtypeStruct((R, D), jnp.float32),
            jax.ShapeDtypeStruct((R, D), jnp.float32),
        ],
    )(deg2, xp, w1)


def _tc_mid(z1, u1, dinv, b1, w2):
    def body(z_ref, u_ref, dinv_ref, b_ref, w2_ref, u2_ref):
        agg = z_ref[0] + z_ref[1] + u_ref[...]
        h1 = jnp.maximum(dinv_ref[...] * agg + b_ref[...], 0.0)
        lin2 = jnp.dot(h1, w2_ref[...], preferred_element_type=jnp.float32)
        u2_ref[...] = dinv_ref[...] * lin2

    return pl.pallas_call(
        body,
        grid=(R // BR,),
        in_specs=[
            pl.BlockSpec((NC, BR, D), lambda i: (0, i, 0)),
            pl.BlockSpec((BR, D), lambda i: (i, 0)),
            pl.BlockSpec((BR, D), lambda i: (i, 0)),
            pl.BlockSpec((1, D), lambda i: (0, 0)),
            pl.BlockSpec((D, D), lambda i: (0, 0)),
        ],
        out_specs=pl.BlockSpec((BR, D), lambda i: (i, 0)),
        out_shape=jax.ShapeDtypeStruct((R, D), jnp.float32),
    )(z1, u1, dinv, b1, w2)


def _tc_final(z2, u2, dinv, b2, w3, b3):
    def body(z_ref, u_ref, dinv_ref, b2_ref, w3_ref, b3_ref, out_ref):
        agg = z_ref[0] + z_ref[1] + u_ref[...]
        h2 = jnp.maximum(dinv_ref[...] * agg + b2_ref[...], 0.0)
        o = jnp.dot(h2, w3_ref[...], preferred_element_type=jnp.float32)
        o = o + b3_ref[...]
        m = jnp.max(o, axis=-1, keepdims=True)
        sh = o - m
        lse = jnp.log(jnp.sum(jnp.exp(sh), axis=-1, keepdims=True))
        out_ref[...] = sh - lse

    return pl.pallas_call(
        body,
        grid=(R // BR,),
        in_specs=[
            pl.BlockSpec((NC, BR, D), lambda i: (0, i, 0)),
            pl.BlockSpec((BR, D), lambda i: (i, 0)),
            pl.BlockSpec((BR, D), lambda i: (i, 0)),
            pl.BlockSpec((1, D), lambda i: (0, 0)),
            pl.BlockSpec((D, 1), lambda i: (0, 0)),
            pl.BlockSpec((1, 1), lambda i: (0, 0)),
        ],
        out_specs=pl.BlockSpec((BR, 1), lambda i: (i, 0)),
        out_shape=jax.ShapeDtypeStruct((R, 1), jnp.float32),
    )(z2, u2, dinv, b2, w3, b3)


def kernel(x, edge_index, W1, b1, W2, b2, W3, b3):
    ei = edge_index.astype(jnp.int32)
    pad = EPAD - E
    src = jnp.concatenate([ei[0], jnp.zeros((pad,), jnp.int32)])
    dst = jnp.concatenate([ei[1], jnp.full((pad,), N, jnp.int32)])
    src3 = src.reshape(NW, K, CH)
    dst3 = dst.reshape(NW, K, CH)
    xp = jnp.pad(x, ((0, R - N), (0, 0)))

    deg2 = _sc_degree(dst3)
    dinv, u1 = _tc_prep(deg2, xp, W1)
    z1 = _sc_aggregate(u1, src3, dst3)
    u2 = _tc_mid(z1, u1, dinv, b1.reshape(1, D), W2)
    z2 = _sc_aggregate(u2, src3, dst3)
    out = _tc_final(z2, u2, dinv, b2.reshape(1, D), W3, b3.reshape(1, 1))
    return out[:N]


# speedup vs baseline: 38.5744x; 1.3153x over previous
import functools

import jax
import jax.numpy as jnp
from jax import lax
from jax.experimental import pallas as pl
from jax.experimental.pallas import tpu as pltpu
from jax.experimental.pallas import tpu_sc as plsc

N = 10000
E = 320000
FIN = 128
D = 16
NC, NS = 2, 16
NW = NC * NS
CH = 128
K = 80
NB = 4
EPAD = NW * K * CH
R = 10240
ZRD = R // NS
BR = 512


def _mesh():
    return plsc.VectorSubcoreMesh(core_axis_name="c", subcore_axis_name="s")


_SC_PARAMS = pltpu.CompilerParams(use_tc_tiling_on_sc=False)


def _sc_degree(dst3):
    @functools.partial(
        pl.kernel,
        mesh=_mesh(),
        out_type=jax.ShapeDtypeStruct((NC, R), jnp.float32),
        compiler_params=_SC_PARAMS,
        scratch_types=[
            pltpu.VMEM((K, CH), jnp.int32),
            pltpu.VMEM((CH,), jnp.float32),
            pltpu.VMEM((ZRD,), jnp.float32),
            pltpu.VMEM_SHARED((R,), jnp.float32),
        ],
    )
    def k(dst_hbm, out_hbm, dst_v, ones_v, zb_v, dtab):
        c = lax.axis_index("c")
        s = lax.axis_index("s")
        w = c * NS + s

        z16 = jnp.zeros((16,), jnp.float32)
        o16 = jnp.ones((16,), jnp.float32)

        def fz(i, carry):
            zb_v[pl.ds(i * 16, 16)] = z16
            return carry

        lax.fori_loop(0, ZRD // 16, fz, 0)

        def fo(i, carry):
            ones_v[pl.ds(i * 16, 16)] = o16
            return carry

        lax.fori_loop(0, CH // 16, fo, 0)

        pltpu.sync_copy(zb_v, dtab.at[pl.ds(s * ZRD, ZRD)])
        plsc.subcore_barrier()

        pltpu.sync_copy(dst_hbm.at[w], dst_v)

        def step(j, carry):
            pltpu.sync_copy(ones_v, dtab.at[dst_v.at[j]], add=True)
            return carry

        lax.fori_loop(0, K, step, 0)
        plsc.subcore_barrier()

        @pl.when(s == 0)
        def _():
            pltpu.sync_copy(dtab, out_hbm.at[c])

    return k(dst3)


def _sc_aggregate(ytab, src3, dst3):
    @functools.partial(
        pl.kernel,
        mesh=_mesh(),
        out_type=jax.ShapeDtypeStruct((NC, R, D), jnp.float32),
        compiler_params=_SC_PARAMS,
        scratch_types=[
            pltpu.VMEM((K, CH), jnp.int32),
            pltpu.VMEM((K, CH), jnp.int32),
            [pltpu.VMEM((CH, D), jnp.float32) for _ in range(NB)],
            pltpu.VMEM((ZRD, D), jnp.float32),
            pltpu.VMEM_SHARED((R, D), jnp.float32),
            [pltpu.SemaphoreType.DMA for _ in range(NB)],
        ],
    )
    def k(ytab_hbm, src_hbm, dst_hbm, out_hbm, src_v, dst_v, bufs, zb, ztab, sems):
        c = lax.axis_index("c")
        s = lax.axis_index("s")
        w = c * NS + s

        zrow = jnp.zeros((D,), jnp.float32)

        def fz(i, carry):
            zb[i, :] = zrow
            return carry

        lax.fori_loop(0, ZRD, fz, 0)
        pltpu.sync_copy(zb, ztab.at[pl.ds(s * ZRD, ZRD)])
        plsc.subcore_barrier()

        pltpu.sync_copy(src_hbm.at[w], src_v)
        pltpu.sync_copy(dst_hbm.at[w], dst_v)

        for b in range(NB):
            pltpu.async_copy(ytab_hbm.at[src_v.at[b]], bufs[b], sems[b])

        def block(i, carry):
            j0 = i * NB
            for b in range(NB):
                j = j0 + b
                pltpu.make_async_copy(
                    ytab_hbm.at[src_v.at[j]], bufs[b], sems[b]
                ).wait()
                pltpu.sync_copy(bufs[b], ztab.at[dst_v.at[j]], add=True)

                @pl.when(j + NB < K)
                def _():
                    pltpu.async_copy(
                        ytab_hbm.at[src_v.at[j + NB]], bufs[b], sems[b]
                    )

            return carry

        lax.fori_loop(0, K // NB, block, 0)
        plsc.subcore_barrier()

        @pl.when(s == 0)
        def _():
            pltpu.sync_copy(ztab, out_hbm.at[c])

    return k(ytab, src3, dst3)


def _tc_prep(deg2, xp, w1):
    def body(deg_ref, x_ref, w1_ref, dinv_ref, u1_ref):
        deg = deg_ref[0, :] + deg_ref[1, :] + 1.0
        dinv = lax.rsqrt(deg)
        lin = jnp.dot(x_ref[...], w1_ref[...], preferred_element_type=jnp.float32)
        dcol = dinv[:, None]
        u1_ref[...] = lin * dcol
        dinv_ref[...] = jnp.broadcast_to(dcol, (BR, D))

    return pl.pallas_call(
        body,
        grid=(R // BR,),
        in_specs=[
            pl.BlockSpec((NC, BR), lambda i: (0, i)),
            pl.BlockSpec((BR, FIN), lambda i: (i, 0)),
            pl.BlockSpec((FIN, D), lambda i: (0, 0)),
        ],
        out_specs=[
            pl.BlockSpec((BR, D), lambda i: (i, 0)),
            pl.BlockSpec((BR, D), lambda i: (i, 0)),
        ],
        out_shape=[
            jax.ShapeDtypeStruct((R, D), jnp.float32),
            jax.ShapeDtypeStruct((R, D), jnp.float32),
        ],
    )(deg2, xp, w1)


def _tc_mid(z1, u1, dinv, b1, w2):
    def body(z_ref, u_ref, dinv_ref, b_ref, w2_ref, u2_ref):
        agg = z_ref[0] + z_ref[1] + u_ref[...]
        h1 = jnp.maximum(dinv_ref[...] * agg + b_ref[...], 0.0)
        lin2 = jnp.dot(h1, w2_ref[...], preferred_element_type=jnp.float32)
        u2_ref[...] = dinv_ref[...] * lin2

    return pl.pallas_call(
        body,
        grid=(R // BR,),
        in_specs=[
            pl.BlockSpec((NC, BR, D), lambda i: (0, i, 0)),
            pl.BlockSpec((BR, D), lambda i: (i, 0)),
            pl.BlockSpec((BR, D), lambda i: (i, 0)),
            pl.BlockSpec((1, D), lambda i: (0, 0)),
            pl.BlockSpec((D, D), lambda i: (0, 0)),
        ],
        out_specs=pl.BlockSpec((BR, D), lambda i: (i, 0)),
        out_shape=jax.ShapeDtypeStruct((R, D), jnp.float32),
    )(z1, u1, dinv, b1, w2)


def _tc_final(z2, u2, dinv, b2, w3, b3):
    def body(z_ref, u_ref, dinv_ref, b2_ref, w3_ref, b3_ref, out_ref):
        agg = z_ref[0] + z_ref[1] + u_ref[...]
        h2 = jnp.maximum(dinv_ref[...] * agg + b2_ref[...], 0.0)
        o = jnp.dot(h2, w3_ref[...], preferred_element_type=jnp.float32)
        o = o + b3_ref[...]
        m = jnp.max(o, axis=-1, keepdims=True)
        sh = o - m
        lse = jnp.log(jnp.sum(jnp.exp(sh), axis=-1, keepdims=True))
        out_ref[...] = sh - lse

    return pl.pallas_call(
        body,
        grid=(R // BR,),
        in_specs=[
            pl.BlockSpec((NC, BR, D), lambda i: (0, i, 0)),
            pl.BlockSpec((BR, D), lambda i: (i, 0)),
            pl.BlockSpec((BR, D), lambda i: (i, 0)),
            pl.BlockSpec((1, D), lambda i: (0, 0)),
            pl.BlockSpec((D, 1), lambda i: (0, 0)),
            pl.BlockSpec((1, 1), lambda i: (0, 0)),
        ],
        out_specs=pl.BlockSpec((BR, 1), lambda i: (i, 0)),
        out_shape=jax.ShapeDtypeStruct((R, 1), jnp.float32),
    )(z2, u2, dinv, b2, w3, b3)


def kernel(x, edge_index, W1, b1, W2, b2, W3, b3):
    ei = edge_index.astype(jnp.int32)
    pad = EPAD - E
    src = jnp.concatenate([ei[0], jnp.zeros((pad,), jnp.int32)])
    dst = jnp.concatenate([ei[1], jnp.full((pad,), N, jnp.int32)])
    src3 = src.reshape(NW, K, CH)
    dst3 = dst.reshape(NW, K, CH)
    xp = jnp.pad(x, ((0, R - N), (0, 0)))

    deg2 = _sc_degree(dst3)
    dinv, u1 = _tc_prep(deg2, xp, W1)
    z1 = _sc_aggregate(u1, src3, dst3)
    u2 = _tc_mid(z1, u1, dinv, b1.reshape(1, D), W2)
    z2 = _sc_aggregate(u2, src3, dst3)
    out = _tc_final(z2, u2, dinv, b2.reshape(1, D), W3, b3.reshape(1, 1))
    return out[:N]


# speedup vs baseline: 55.6921x; 1.4438x over previous
import functools

import jax
import jax.numpy as jnp
from jax import lax
from jax.experimental import pallas as pl
from jax.experimental.pallas import tpu as pltpu
from jax.experimental.pallas import tpu_sc as plsc

N = 10000
E = 320000
FIN = 128
D = 16
NC, NS = 2, 16
NW = NC * NS
CH = 128
TCH = E // CH
NB = 4
KMAX = 80
DR = 10240
ZRD = N // NS
BR = 1000


def _mesh():
    return plsc.VectorSubcoreMesh(core_axis_name="c", subcore_axis_name="s")


_SC_PARAMS = pltpu.CompilerParams(use_tc_tiling_on_sc=False)


def _worker_chunks(w):
    big = (w < 17).astype(jnp.int32)
    off = 76 * w + 4 * jnp.minimum(w, 17)
    kw = 76 + 4 * big
    return off, kw


def _load_idx(idx_hbm, idx_v, off, w):
    pltpu.sync_copy(idx_hbm.at[pl.ds(off, 76)], idx_v.at[pl.ds(0, 76)])

    @pl.when(w < 17)
    def _():
        pltpu.sync_copy(idx_hbm.at[pl.ds(off + 76, 4)], idx_v.at[pl.ds(76, 4)])


def _sc_degree(dst2):
    @functools.partial(
        pl.kernel,
        mesh=_mesh(),
        out_type=jax.ShapeDtypeStruct((NC, DR), jnp.float32),
        compiler_params=_SC_PARAMS,
        scratch_types=[
            pltpu.VMEM((KMAX, CH), jnp.int32),
            pltpu.VMEM((CH,), jnp.float32),
            pltpu.VMEM((DR // NS,), jnp.float32),
            pltpu.VMEM_SHARED((DR,), jnp.float32),
        ],
    )
    def k(dst_hbm, out_hbm, dst_v, ones_v, zb_v, dtab):
        c = lax.axis_index("c")
        s = lax.axis_index("s")
        w = c * NS + s

        z16 = jnp.zeros((16,), jnp.float32)
        o16 = jnp.ones((16,), jnp.float32)

        def fz(i, carry):
            zb_v[pl.ds(i * 16, 16)] = z16
            return carry

        lax.fori_loop(0, DR // NS // 16, fz, 0)

        def fo(i, carry):
            ones_v[pl.ds(i * 16, 16)] = o16
            return carry

        lax.fori_loop(0, CH // 16, fo, 0)

        pltpu.sync_copy(zb_v, dtab.at[pl.ds(s * (DR // NS), DR // NS)])
        plsc.subcore_barrier()

        off, kw = _worker_chunks(w)
        _load_idx(dst_hbm, dst_v, off, w)

        def step(j, carry):
            pltpu.sync_copy(ones_v, dtab.at[dst_v.at[j]], add=True)
            return carry

        lax.fori_loop(0, kw, step, 0)
        plsc.subcore_barrier()

        @pl.when(s == 0)
        def _():
            pltpu.sync_copy(dtab, out_hbm.at[c])

    return k(dst2)


def _sc_aggregate(ytab, src2, dst2):
    @functools.partial(
        pl.kernel,
        mesh=_mesh(),
        out_type=jax.ShapeDtypeStruct((NC, N, D), jnp.float32),
        compiler_params=_SC_PARAMS,
        scratch_types=[
            pltpu.VMEM((KMAX, CH), jnp.int32),
            pltpu.VMEM((KMAX, CH), jnp.int32),
            [pltpu.VMEM((CH, D), jnp.float32) for _ in range(NB)],
            pltpu.VMEM((ZRD, D), jnp.float32),
            pltpu.VMEM_SHARED((N, D), jnp.float32),
            [pltpu.SemaphoreType.DMA for _ in range(NB)],
        ],
    )
    def k(ytab_hbm, src_hbm, dst_hbm, out_hbm, src_v, dst_v, bufs, zb, ztab, sems):
        c = lax.axis_index("c")
        s = lax.axis_index("s")
        w = c * NS + s

        zrow = jnp.zeros((D,), jnp.float32)

        def fz(i, carry):
            zb[i, :] = zrow
            return carry

        lax.fori_loop(0, ZRD, fz, 0)
        pltpu.sync_copy(zb, ztab.at[pl.ds(s * ZRD, ZRD)])
        plsc.subcore_barrier()

        off, kw = _worker_chunks(w)
        _load_idx(src_hbm, src_v, off, w)
        _load_idx(dst_hbm, dst_v, off, w)

        for b in range(NB):
            pltpu.async_copy(ytab_hbm.at[src_v.at[b]], bufs[b], sems[b])

        def block(i, carry):
            j0 = i * NB
            for b in range(NB):
                j = j0 + b
                pltpu.make_async_copy(
                    ytab_hbm.at[src_v.at[j]], bufs[b], sems[b]
                ).wait()
                pltpu.sync_copy(bufs[b], ztab.at[dst_v.at[j]], add=True)

                @pl.when(j + NB < kw)
                def _():
                    pltpu.async_copy(
                        ytab_hbm.at[src_v.at[j + NB]], bufs[b], sems[b]
                    )

            return carry

        lax.fori_loop(0, kw // NB, block, 0)
        plsc.subcore_barrier()

        @pl.when(s == 0)
        def _():
            pltpu.sync_copy(ztab, out_hbm.at[c])

    return k(ytab, src2, dst2)


def _tc_prep(degt, x, w1):
    def body(deg_ref, x_ref, w1_ref, dinv_ref, u1_ref):
        deg = deg_ref[0, :, 0] + deg_ref[0, :, 1] + 1.0
        dinv = lax.rsqrt(deg)
        lin = jnp.dot(x_ref[...], w1_ref[...], preferred_element_type=jnp.float32)
        dcol = dinv[:, None]
        u1_ref[...] = lin * dcol
        dinv_ref[...] = jnp.broadcast_to(dcol, (BR, D))

    return pl.pallas_call(
        body,
        grid=(N // BR,),
        in_specs=[
            pl.BlockSpec((1, BR, NC), lambda i: (i, 0, 0)),
            pl.BlockSpec((BR, FIN), lambda i: (i, 0)),
            pl.BlockSpec((FIN, D), lambda i: (0, 0)),
        ],
        out_specs=[
            pl.BlockSpec((BR, D), lambda i: (i, 0)),
            pl.BlockSpec((BR, D), lambda i: (i, 0)),
        ],
        out_shape=[
            jax.ShapeDtypeStruct((N, D), jnp.float32),
            jax.ShapeDtypeStruct((N, D), jnp.float32),
        ],
    )(degt, x, w1)


def _tc_mid(z1, u1, dinv, b1, w2):
    def body(z_ref, u_ref, dinv_ref, b_ref, w2_ref, u2_ref):
        agg = z_ref[0] + z_ref[1] + u_ref[...]
        h1 = jnp.maximum(dinv_ref[...] * agg + b_ref[...], 0.0)
        lin2 = jnp.dot(h1, w2_ref[...], preferred_element_type=jnp.float32)
        u2_ref[...] = dinv_ref[...] * lin2

    return pl.pallas_call(
        body,
        grid=(N // BR,),
        in_specs=[
            pl.BlockSpec((NC, BR, D), lambda i: (0, i, 0)),
            pl.BlockSpec((BR, D), lambda i: (i, 0)),
            pl.BlockSpec((BR, D), lambda i: (i, 0)),
            pl.BlockSpec((1, D), lambda i: (0, 0)),
            pl.BlockSpec((D, D), lambda i: (0, 0)),
        ],
        out_specs=pl.BlockSpec((BR, D), lambda i: (i, 0)),
        out_shape=jax.ShapeDtypeStruct((N, D), jnp.float32),
    )(z1, u1, dinv, b1, w2)


def _tc_final(z2, u2, dinv, b2, w3, b3):
    def body(z_ref, u_ref, dinv_ref, b2_ref, w3_ref, b3_ref, out_ref):
        agg = z_ref[0] + z_ref[1] + u_ref[...]
        h2 = jnp.maximum(dinv_ref[...] * agg + b2_ref[...], 0.0)
        o = jnp.dot(h2, w3_ref[...], preferred_element_type=jnp.float32)
        o = o + b3_ref[...]
        m = jnp.max(o, axis=-1, keepdims=True)
        sh = o - m
        lse = jnp.log(jnp.sum(jnp.exp(sh), axis=-1, keepdims=True))
        out_ref[...] = sh - lse

    return pl.pallas_call(
        body,
        grid=(N // BR,),
        in_specs=[
            pl.BlockSpec((NC, BR, D), lambda i: (0, i, 0)),
            pl.BlockSpec((BR, D), lambda i: (i, 0)),
            pl.BlockSpec((BR, D), lambda i: (i, 0)),
            pl.BlockSpec((1, D), lambda i: (0, 0)),
            pl.BlockSpec((D, 1), lambda i: (0, 0)),
            pl.BlockSpec((1, 1), lambda i: (0, 0)),
        ],
        out_specs=pl.BlockSpec((BR, 1), lambda i: (i, 0)),
        out_shape=jax.ShapeDtypeStruct((N, 1), jnp.float32),
    )(z2, u2, dinv, b2, w3, b3)


def kernel(x, edge_index, W1, b1, W2, b2, W3, b3):
    ei = edge_index.astype(jnp.int32)
    src2 = ei[0].reshape(TCH, CH)
    dst2 = ei[1].reshape(TCH, CH)

    deg2 = _sc_degree(dst2)
    degt = deg2[:, :N].T.reshape(N // BR, BR, NC)
    dinv, u1 = _tc_prep(degt, x, W1)
    z1 = _sc_aggregate(u1, src2, dst2)
    u2 = _tc_mid(z1, u1, dinv, b1.reshape(1, D), W2)
    z2 = _sc_aggregate(u2, src2, dst2)
    return _tc_final(z2, u2, dinv, b2.reshape(1, D), W3, b3.reshape(1, 1))


# speedup vs baseline: 69.3254x; 1.2448x over previous
import functools

import jax
import jax.numpy as jnp
from jax import lax
from jax.experimental import pallas as pl
from jax.experimental.pallas import tpu as pltpu
from jax.experimental.pallas import tpu_sc as plsc

N = 10000
E = 320000
FIN = 128
D = 16
NC, NS = 2, 16
NW = NC * NS
CH = 128
TCH = E // CH
NB = 4
KMAX = 80
DR = 10240
ZRD = N // NS
NV = N // 8


def _mesh():
    return plsc.VectorSubcoreMesh(core_axis_name="c", subcore_axis_name="s")


_SC_PARAMS = pltpu.CompilerParams(use_tc_tiling_on_sc=False)


def _worker_chunks(w):
    big = (w < 17).astype(jnp.int32)
    off = 76 * w + 4 * jnp.minimum(w, 17)
    kw = 76 + 4 * big
    return off, kw


def _load_idx(idx_hbm, idx_v, off, w):
    pltpu.sync_copy(idx_hbm.at[pl.ds(off, 76)], idx_v.at[pl.ds(0, 76)])

    @pl.when(w < 17)
    def _():
        pltpu.sync_copy(idx_hbm.at[pl.ds(off + 76, 4)], idx_v.at[pl.ds(76, 4)])


def _sc_degree(dst2):
    @functools.partial(
        pl.kernel,
        mesh=_mesh(),
        out_type=jax.ShapeDtypeStruct((NC, DR, D), jnp.float32),
        compiler_params=_SC_PARAMS,
        scratch_types=[
            pltpu.VMEM((KMAX, CH), jnp.int32),
            pltpu.VMEM((CH, D), jnp.float32),
            pltpu.VMEM((DR // NS, D), jnp.float32),
            pltpu.VMEM_SHARED((DR, D), jnp.float32),
        ],
    )
    def k(dst_hbm, out_hbm, dst_v, ones_v, zb_v, dtab):
        c = lax.axis_index("c")
        s = lax.axis_index("s")
        w = c * NS + s

        z16 = jnp.zeros((D,), jnp.float32)
        o16 = jnp.ones((D,), jnp.float32)

        def fz(i, carry):
            zb_v[i, :] = z16
            return carry

        lax.fori_loop(0, DR // NS, fz, 0)

        def fo(i, carry):
            ones_v[i, :] = o16
            return carry

        lax.fori_loop(0, CH, fo, 0)

        pltpu.sync_copy(zb_v, dtab.at[pl.ds(s * (DR // NS), DR // NS)])
        plsc.subcore_barrier()

        off, kw = _worker_chunks(w)
        _load_idx(dst_hbm, dst_v, off, w)

        def step(j, carry):
            pltpu.sync_copy(ones_v, dtab.at[dst_v.at[j]], add=True)
            return carry

        lax.fori_loop(0, kw, step, 0)
        plsc.subcore_barrier()

        @pl.when(s == 0)
        def _():
            pltpu.sync_copy(dtab, out_hbm.at[c])

    return k(dst2)


def _sc_aggregate(ytab, src2, dst2):
    @functools.partial(
        pl.kernel,
        mesh=_mesh(),
        out_type=jax.ShapeDtypeStruct((NC, N, D), jnp.float32),
        compiler_params=_SC_PARAMS,
        scratch_types=[
            pltpu.VMEM((KMAX, CH), jnp.int32),
            pltpu.VMEM((KMAX, CH), jnp.int32),
            [pltpu.VMEM((CH, D), jnp.float32) for _ in range(NB)],
            pltpu.VMEM((ZRD, D), jnp.float32),
            pltpu.VMEM_SHARED((N, D), jnp.float32),
            [pltpu.SemaphoreType.DMA for _ in range(NB)],
        ],
    )
    def k(ytab_hbm, src_hbm, dst_hbm, out_hbm, src_v, dst_v, bufs, zb, ztab, sems):
        c = lax.axis_index("c")
        s = lax.axis_index("s")
        w = c * NS + s

        zrow = jnp.zeros((D,), jnp.float32)

        def fz(i, carry):
            zb[i, :] = zrow
            return carry

        lax.fori_loop(0, ZRD, fz, 0)
        pltpu.sync_copy(zb, ztab.at[pl.ds(s * ZRD, ZRD)])
        plsc.subcore_barrier()

        off, kw = _worker_chunks(w)
        _load_idx(src_hbm, src_v, off, w)
        _load_idx(dst_hbm, dst_v, off, w)

        for b in range(NB):
            pltpu.async_copy(ytab_hbm.at[src_v.at[b]], bufs[b], sems[b])

        def block(i, carry):
            j0 = i * NB
            for b in range(NB):
                j = j0 + b
                pltpu.make_async_copy(
                    ytab_hbm.at[src_v.at[j]], bufs[b], sems[b]
                ).wait()
                pltpu.sync_copy(bufs[b], ztab.at[dst_v.at[j]], add=True)

                @pl.when(j + NB < kw)
                def _():
                    pltpu.async_copy(
                        ytab_hbm.at[src_v.at[j + NB]], bufs[b], sems[b]
                    )

            return carry

        lax.fori_loop(0, kw // NB, block, 0)
        plsc.subcore_barrier()

        @pl.when(s == 0)
        def _():
            pltpu.sync_copy(ztab, out_hbm.at[c])

    return k(ytab, src2, dst2)


def _tc_prep(degv, x, w1):
    def body(deg_ref, x_ref, w1_ref, dinv_ref, u1_ref):
        deg_v = deg_ref[0, :NV] + deg_ref[1, :NV] + 1.0
        dinv_v = lax.rsqrt(deg_v)
        parts = [
            jnp.dot(x_ref[pl.ds(NV * a, NV), :], w1_ref[...],
                    preferred_element_type=jnp.float32)
            for a in range(8)
        ]
        lin_v = jnp.concatenate(parts, axis=1)
        dinv_ref[...] = dinv_v
        u1_ref[...] = lin_v * dinv_v

    return pl.pallas_call(
        body,
        out_shape=[
            jax.ShapeDtypeStruct((NV, 128), jnp.float32),
            jax.ShapeDtypeStruct((NV, 128), jnp.float32),
        ],
    )(degv, x, w1)


def _tc_mid(z1v, u1v, dinvv, b1t, w2e):
    def body(z_ref, u_ref, dinv_ref, b_ref, w2e_ref, u2_ref):
        agg = z_ref[0] + z_ref[1] + u_ref[...]
        h1 = jnp.maximum(dinv_ref[...] * agg + b_ref[...], 0.0)
        lin2 = jnp.dot(h1, w2e_ref[...], preferred_element_type=jnp.float32)
        u2_ref[...] = dinv_ref[...] * lin2

    return pl.pallas_call(
        body,
        out_shape=jax.ShapeDtypeStruct((NV, 128), jnp.float32),
    )(z1v, u1v, dinvv, b1t, w2e)


def _tc_final(z2v, u2v, dinvv, b2t, w3e, b3):
    def body(z_ref, u_ref, dinv_ref, b2_ref, w3e_ref, b3_ref, out_ref):
        agg = z_ref[0] + z_ref[1] + u_ref[...]
        h2 = jnp.maximum(dinv_ref[...] * agg + b2_ref[...], 0.0)
        o = jnp.dot(h2, w3e_ref[...], preferred_element_type=jnp.float32)
        o = o + b3_ref[0, 0]
        sh = o - o
        out_ref[...] = sh - jnp.log(jnp.exp(sh))

    return pl.pallas_call(
        body,
        out_shape=jax.ShapeDtypeStruct((NV, 8), jnp.float32),
    )(z2v, u2v, dinvv, b2t, w3e, b3)


def kernel(x, edge_index, W1, b1, W2, b2, W3, b3):
    ei = edge_index.astype(jnp.int32)
    eip = 8 * (ei % NV) + ei // NV
    src2 = eip[0].reshape(TCH, CH)
    dst2 = eip[1].reshape(TCH, CH)

    e8 = jnp.eye(8, dtype=jnp.float32)
    w2e = jnp.einsum("ac,kf->akcf", e8, W2).reshape(128, 128)
    w3e = (e8[:, None, :] * W3[None, :, 0, None]).reshape(128, 8)
    b1t = jnp.tile(b1, 8)[None, :]
    b2t = jnp.tile(b2, 8)[None, :]

    deg2 = _sc_degree(dst2)
    degv = deg2.reshape(NC, DR * D // 128, 128)
    dinvv, u1v = _tc_prep(degv, x, W1)
    z1 = _sc_aggregate(u1v.reshape(N, D), src2, dst2)
    u2v = _tc_mid(z1.reshape(NC, NV, 128), u1v, dinvv, b1t, w2e)
    z2 = _sc_aggregate(u2v.reshape(N, D), src2, dst2)
    out = _tc_final(z2.reshape(NC, NV, 128), u2v, dinvv, b2t, w3e,
                    b3.reshape(1, 1))
    return out.reshape(N, 1)


# speedup vs baseline: 76.8708x; 1.1088x over previous
import functools

import jax
import jax.numpy as jnp
from jax import lax
from jax.experimental import pallas as pl
from jax.experimental.pallas import tpu as pltpu
from jax.experimental.pallas import tpu_sc as plsc

N = 10000
E = 320000
FIN = 128
D = 16
NC, NS = 2, 16
NW = NC * NS
CH = 128
TCH = E // CH
NB = 4
KMAX = 80
DR = 10240
ZRD = N // NS
NV = N // 8


def _mesh():
    return plsc.VectorSubcoreMesh(core_axis_name="c", subcore_axis_name="s")


_SC_PARAMS = pltpu.CompilerParams(use_tc_tiling_on_sc=False)


def _worker_chunks(w):
    big = (w < 17).astype(jnp.int32)
    off = 76 * w + 4 * jnp.minimum(w, 17)
    kw = 76 + 4 * big
    return off, kw


def _load_idx(idx_hbm, plane, idx_v, off, w):
    pltpu.sync_copy(idx_hbm.at[plane, pl.ds(off, 76)], idx_v.at[pl.ds(0, 76)])

    @pl.when(w < 17)
    def _():
        pltpu.sync_copy(
            idx_hbm.at[plane, pl.ds(off + 76, 4)], idx_v.at[pl.ds(76, 4)]
        )


def _tc_edgeprep(ei):
    def body(ei_ref, out_ref):
        v = ei_ref[...]
        vp = 8 * lax.rem(v, NV) + lax.div(v, NV)
        out_ref[...] = vp.reshape(2, TCH, CH)

    return pl.pallas_call(
        body,
        out_shape=jax.ShapeDtypeStruct((2, TCH, CH), jnp.int32),
    )(ei)


def _sc_degree(eip2):
    @functools.partial(
        pl.kernel,
        mesh=_mesh(),
        out_type=jax.ShapeDtypeStruct((NC, DR, D), jnp.float32),
        compiler_params=_SC_PARAMS,
        scratch_types=[
            pltpu.VMEM((KMAX, CH), jnp.int32),
            pltpu.VMEM((CH, D), jnp.float32),
            pltpu.VMEM((DR // NS, D), jnp.float32),
            pltpu.VMEM_SHARED((DR, D), jnp.float32),
        ],
    )
    def k(ei_hbm, out_hbm, dst_v, ones_v, zb_v, dtab):
        c = lax.axis_index("c")
        s = lax.axis_index("s")
        w = c * NS + s

        z16 = jnp.zeros((D,), jnp.float32)
        o16 = jnp.ones((D,), jnp.float32)

        def fz(i, carry):
            zb_v[i, :] = z16
            return carry

        lax.fori_loop(0, DR // NS, fz, 0)

        def fo(i, carry):
            ones_v[i, :] = o16
            return carry

        lax.fori_loop(0, CH, fo, 0)

        pltpu.sync_copy(zb_v, dtab.at[pl.ds(s * (DR // NS), DR // NS)])
        plsc.subcore_barrier()

        off, kw = _worker_chunks(w)
        _load_idx(ei_hbm, 1, dst_v, off, w)

        def step(j, carry):
            pltpu.sync_copy(ones_v, dtab.at[dst_v.at[j]], add=True)
            return carry

        lax.fori_loop(0, kw, step, 0)
        plsc.subcore_barrier()

        @pl.when(s == 0)
        def _():
            pltpu.sync_copy(dtab, out_hbm.at[c])

    return k(eip2)


def _sc_aggregate(ytab, eip2):
    @functools.partial(
        pl.kernel,
        mesh=_mesh(),
        out_type=jax.ShapeDtypeStruct((NC, N, D), jnp.float32),
        compiler_params=_SC_PARAMS,
        scratch_types=[
            pltpu.VMEM((KMAX, CH), jnp.int32),
            pltpu.VMEM((KMAX, CH), jnp.int32),
            [pltpu.VMEM((CH, D), jnp.float32) for _ in range(NB)],
            pltpu.VMEM((ZRD, D), jnp.float32),
            pltpu.VMEM_SHARED((N, D), jnp.float32),
            [pltpu.SemaphoreType.DMA for _ in range(NB)],
            [pltpu.SemaphoreType.DMA for _ in range(NB)],
        ],
    )
    def k(ytab_hbm, ei_hbm, out_hbm, src_v, dst_v, bufs, zb, ztab, gsems, ssems):
        c = lax.axis_index("c")
        s = lax.axis_index("s")
        w = c * NS + s

        zrow = jnp.zeros((D,), jnp.float32)

        def fz(i, carry):
            zb[i, :] = zrow
            return carry

        lax.fori_loop(0, ZRD, fz, 0)
        pltpu.sync_copy(zb, ztab.at[pl.ds(s * ZRD, ZRD)])
        plsc.subcore_barrier()

        off, kw = _worker_chunks(w)
        _load_idx(ei_hbm, 0, src_v, off, w)
        _load_idx(ei_hbm, 1, dst_v, off, w)

        for b in range(NB):
            pltpu.async_copy(ytab_hbm.at[src_v.at[b]], bufs[b], gsems[b])

        def block(i, carry):
            j0 = i * NB
            for b in range(NB):
                j = j0 + b
                pltpu.make_async_copy(
                    ytab_hbm.at[src_v.at[j]], bufs[b], gsems[b]
                ).wait()
                pltpu.async_copy(
                    bufs[b], ztab.at[dst_v.at[j]], ssems[b], add=True
                )
            for b in range(NB):
                j = j0 + b
                pltpu.make_async_copy(
                    bufs[b], ztab.at[dst_v.at[j]], ssems[b]
                ).wait()

                @pl.when(j + NB < kw)
                def _():
                    pltpu.async_copy(
                        ytab_hbm.at[src_v.at[j + NB]], bufs[b], gsems[b]
                    )

            return carry

        lax.fori_loop(0, kw // NB, block, 0)
        plsc.subcore_barrier()

        @pl.when(s == 0)
        def _():
            pltpu.sync_copy(ztab, out_hbm.at[c])

    return k(ytab, eip2)


def _tc_prep(degv, x, w1):
    def body(deg_ref, x_ref, w1_ref, dinv_ref, u1_ref):
        deg_v = deg_ref[0, :NV] + deg_ref[1, :NV] + 1.0
        dinv_v = lax.rsqrt(deg_v)
        parts = [
            jnp.dot(x_ref[pl.ds(NV * a, NV), :], w1_ref[...],
                    preferred_element_type=jnp.float32)
            for a in range(8)
        ]
        lin_v = jnp.concatenate(parts, axis=1)
        dinv_ref[...] = dinv_v
        u1_ref[...] = lin_v * dinv_v

    return pl.pallas_call(
        body,
        out_shape=[
            jax.ShapeDtypeStruct((NV, 128), jnp.float32),
            jax.ShapeDtypeStruct((NV, 128), jnp.float32),
        ],
    )(degv, x, w1)


def _tc_mid(z1v, u1v, dinvv, b1t, w2e):
    def body(z_ref, u_ref, dinv_ref, b_ref, w2e_ref, u2_ref):
        agg = z_ref[0] + z_ref[1] + u_ref[...]
        h1 = jnp.maximum(dinv_ref[...] * agg + b_ref[...], 0.0)
        lin2 = jnp.dot(h1, w2e_ref[...], preferred_element_type=jnp.float32)
        u2_ref[...] = dinv_ref[...] * lin2

    return pl.pallas_call(
        body,
        out_shape=jax.ShapeDtypeStruct((NV, 128), jnp.float32),
    )(z1v, u1v, dinvv, b1t, w2e)


def _tc_final(z2v, u2v, dinvv, b2t, w3e, b3):
    def body(z_ref, u_ref, dinv_ref, b2_ref, w3e_ref, b3_ref, out_ref):
        agg = z_ref[0] + z_ref[1] + u_ref[...]
        h2 = jnp.maximum(dinv_ref[...] * agg + b2_ref[...], 0.0)
        o = jnp.dot(h2, w3e_ref[...], preferred_element_type=jnp.float32)
        o = o + b3_ref[0, 0]
        sh = o - o
        out_ref[...] = sh - jnp.log(jnp.exp(sh))

    return pl.pallas_call(
        body,
        out_shape=jax.ShapeDtypeStruct((NV, 8), jnp.float32),
    )(z2v, u2v, dinvv, b2t, w3e, b3)


def kernel(x, edge_index, W1, b1, W2, b2, W3, b3):
    ei = edge_index.astype(jnp.int32)
    eip2 = _tc_edgeprep(ei)

    e8 = jnp.eye(8, dtype=jnp.float32)
    w2e = jnp.einsum("ac,kf->akcf", e8, W2).reshape(128, 128)
    w3e = (e8[:, None, :] * W3[None, :, 0, None]).reshape(128, 8)
    b1t = jnp.tile(b1, 8)[None, :]
    b2t = jnp.tile(b2, 8)[None, :]

    deg2 = _sc_degree(eip2)
    degv = deg2.reshape(NC, DR * D // 128, 128)
    dinvv, u1v = _tc_prep(degv, x, W1)
    z1 = _sc_aggregate(u1v.reshape(N, D), eip2)
    u2v = _tc_mid(z1.reshape(NC, NV, 128), u1v, dinvv, b1t, w2e)
    z2 = _sc_aggregate(u2v.reshape(N, D), eip2)
    out = _tc_final(z2.reshape(NC, NV, 128), u2v, dinvv, b2t, w3e,
                    b3.reshape(1, 1))
    return out.reshape(N, 1)


# speedup vs baseline: 82.7694x; 1.0767x over previous
import functools

import jax
import jax.numpy as jnp
from jax import lax
from jax.experimental import pallas as pl
from jax.experimental.pallas import tpu as pltpu
from jax.experimental.pallas import tpu_sc as plsc

N = 10000
E = 320000
FIN = 128
D = 16
NC, NS = 2, 16
NW = NC * NS
CH = 128
TCH = E // CH
NB = 4
KMAX = 80
DR = 10240
ZRD = N // NS
NV = N // 8


def _mesh():
    return plsc.VectorSubcoreMesh(core_axis_name="c", subcore_axis_name="s")


_SC_PARAMS = pltpu.CompilerParams(use_tc_tiling_on_sc=False)


def _worker_chunks(w):
    big = (w < 17).astype(jnp.int32)
    off = 76 * w + 4 * jnp.minimum(w, 17)
    kw = 76 + 4 * big
    return off, kw


def _load_idx(idx_hbm, plane, idx_v, off, w):
    base = plane * TCH + off
    pltpu.sync_copy(idx_hbm.at[pl.ds(base, 76)], idx_v.at[pl.ds(0, 76)])

    @pl.when(w < 17)
    def _():
        pltpu.sync_copy(idx_hbm.at[pl.ds(base + 76, 4)], idx_v.at[pl.ds(76, 4)])


def _tc_edgeprep(ei):
    def body(ei_ref, out_ref):
        v = ei_ref[...]
        vp = 8 * lax.rem(v, NV) + lax.div(v, NV)
        out_ref[...] = vp.reshape(2 * TCH, CH)

    return pl.pallas_call(
        body,
        out_shape=jax.ShapeDtypeStruct((2 * TCH, CH), jnp.int32),
    )(ei)


def _sc_degree(eip2):
    @functools.partial(
        pl.kernel,
        mesh=_mesh(),
        out_type=jax.ShapeDtypeStruct((NC, DR, D), jnp.float32),
        compiler_params=_SC_PARAMS,
        scratch_types=[
            pltpu.VMEM((KMAX, CH), jnp.int32),
            pltpu.VMEM((CH, D), jnp.float32),
            pltpu.VMEM((DR // NS, D), jnp.float32),
            pltpu.VMEM_SHARED((DR, D), jnp.float32),
        ],
    )
    def k(ei_hbm, out_hbm, dst_v, ones_v, zb_v, dtab):
        c = lax.axis_index("c")
        s = lax.axis_index("s")
        w = c * NS + s

        z16 = jnp.zeros((D,), jnp.float32)
        o16 = jnp.ones((D,), jnp.float32)

        def fz(i, carry):
            zb_v[i, :] = z16
            return carry

        lax.fori_loop(0, DR // NS, fz, 0)

        def fo(i, carry):
            ones_v[i, :] = o16
            return carry

        lax.fori_loop(0, CH, fo, 0)

        pltpu.sync_copy(zb_v, dtab.at[pl.ds(s * (DR // NS), DR // NS)])
        plsc.subcore_barrier()

        off, kw = _worker_chunks(w)
        _load_idx(ei_hbm, 1, dst_v, off, w)

        def step(j, carry):
            pltpu.sync_copy(ones_v, dtab.at[dst_v.at[j]], add=True)
            return carry

        lax.fori_loop(0, kw, step, 0)
        plsc.subcore_barrier()

        @pl.when(s == 0)
        def _():
            pltpu.sync_copy(dtab, out_hbm.at[c])

    return k(eip2)


def _sc_aggregate(ytab, eip2):
    @functools.partial(
        pl.kernel,
        mesh=_mesh(),
        out_type=jax.ShapeDtypeStruct((NC, N, D), jnp.float32),
        compiler_params=_SC_PARAMS,
        scratch_types=[
            pltpu.VMEM((KMAX, CH), jnp.int32),
            pltpu.VMEM((KMAX, CH), jnp.int32),
            [pltpu.VMEM((CH, D), jnp.float32) for _ in range(NB)],
            pltpu.VMEM((ZRD, D), jnp.float32),
            pltpu.VMEM_SHARED((N, D), jnp.float32),
            [pltpu.SemaphoreType.DMA for _ in range(NB)],
        ],
    )
    def k(ytab_hbm, ei_hbm, out_hbm, src_v, dst_v, bufs, zb, ztab, gsems):
        c = lax.axis_index("c")
        s = lax.axis_index("s")
        w = c * NS + s

        zrow = jnp.zeros((D,), jnp.float32)

        def fz(i, carry):
            zb[i, :] = zrow
            return carry

        lax.fori_loop(0, ZRD, fz, 0)
        pltpu.sync_copy(zb, ztab.at[pl.ds(s * ZRD, ZRD)])
        plsc.subcore_barrier()

        off, kw = _worker_chunks(w)
        _load_idx(ei_hbm, 0, src_v, off, w)
        _load_idx(ei_hbm, 1, dst_v, off, w)

        for b in range(NB):
            pltpu.async_copy(ytab_hbm.at[src_v.at[b]], bufs[b], gsems[b])

        def block(i, carry):
            j0 = i * NB
            for b in range(NB):
                j = j0 + b
                pltpu.make_async_copy(
                    ytab_hbm.at[src_v.at[j]], bufs[b], gsems[b]
                ).wait()
                pltpu.sync_copy(bufs[b], ztab.at[dst_v.at[j]], add=True)

                @pl.when(j + NB < kw)
                def _():
                    pltpu.async_copy(
                        ytab_hbm.at[src_v.at[j + NB]], bufs[b], gsems[b]
                    )

            return carry

        lax.fori_loop(0, kw // NB, block, 0)
        plsc.subcore_barrier()

        @pl.when(s == 0)
        def _():
            pltpu.sync_copy(ztab, out_hbm.at[c])

    return k(ytab, eip2)


def _tc_lin(x, w1):
    def body(x_ref, w1_ref, lin_ref):
        parts = [
            jnp.dot(x_ref[pl.ds(NV * a, NV), :], w1_ref[...],
                    preferred_element_type=jnp.float32)
            for a in range(8)
        ]
        lin_ref[...] = jnp.concatenate(parts, axis=1)

    return pl.pallas_call(
        body,
        out_shape=jax.ShapeDtypeStruct((NV, 128), jnp.float32),
    )(x, w1)


def _tc_scale(degv, linv):
    def body(deg_ref, lin_ref, dinv_ref, u1_ref):
        deg_v = deg_ref[0, :NV] + deg_ref[1, :NV] + 1.0
        dinv_v = lax.rsqrt(deg_v)
        dinv_ref[...] = dinv_v
        u1_ref[...] = lin_ref[...] * dinv_v

    return pl.pallas_call(
        body,
        out_shape=[
            jax.ShapeDtypeStruct((NV, 128), jnp.float32),
            jax.ShapeDtypeStruct((NV, 128), jnp.float32),
        ],
    )(degv, linv)


def _tc_mid(z1v, u1v, dinvv, b1t, w2e):
    def body(z_ref, u_ref, dinv_ref, b_ref, w2e_ref, u2_ref):
        agg = z_ref[0] + z_ref[1] + u_ref[...]
        h1 = jnp.maximum(dinv_ref[...] * agg + b_ref[...], 0.0)
        lin2 = jnp.dot(h1, w2e_ref[...], preferred_element_type=jnp.float32)
        u2_ref[...] = dinv_ref[...] * lin2

    return pl.pallas_call(
        body,
        out_shape=jax.ShapeDtypeStruct((NV, 128), jnp.float32),
    )(z1v, u1v, dinvv, b1t, w2e)


def _tc_final(z2v, u2v, dinvv, b2t, w3e, b3):
    def body(z_ref, u_ref, dinv_ref, b2_ref, w3e_ref, b3_ref, out_ref):
        agg = z_ref[0] + z_ref[1] + u_ref[...]
        h2 = jnp.maximum(dinv_ref[...] * agg + b2_ref[...], 0.0)
        o = jnp.dot(h2, w3e_ref[...], preferred_element_type=jnp.float32)
        o = o + b3_ref[0, 0]
        sh = o - o
        out_ref[...] = sh - jnp.log(jnp.exp(sh))

    return pl.pallas_call(
        body,
        out_shape=jax.ShapeDtypeStruct((NV, 8), jnp.float32),
    )(z2v, u2v, dinvv, b2t, w3e, b3)


def kernel(x, edge_index, W1, b1, W2, b2, W3, b3):
    ei = edge_index.astype(jnp.int32)
    eip2 = _tc_edgeprep(ei)

    e8 = jnp.eye(8, dtype=jnp.float32)
    w2e = jnp.einsum("ac,kf->akcf", e8, W2).reshape(128, 128)
    w3e = (e8[:, None, :] * W3[None, :, 0, None]).reshape(128, 8)
    b1t = jnp.tile(b1, 8)[None, :]
    b2t = jnp.tile(b2, 8)[None, :]

    linv = _tc_lin(x, W1)
    deg2 = _sc_degree(eip2)
    degv = deg2.reshape(NC, DR * D // 128, 128)
    dinvv, u1v = _tc_scale(degv, linv)
    z1 = _sc_aggregate(u1v.reshape(N, D), eip2)
    u2v = _tc_mid(z1.reshape(NC, NV, 128), u1v, dinvv, b1t, w2e)
    z2 = _sc_aggregate(u2v.reshape(N, D), eip2)
    out = _tc_final(z2.reshape(NC, NV, 128), u2v, dinvv, b2t, w3e,
                    b3.reshape(1, 1))
    return out.reshape(N, 1)
